# Initial kernel scaffold; baseline (speedup 1.0000x reference)
import jax
import jax.numpy as jnp
from jax.experimental import pallas as pl


def kernel(f, g, memory_bank, update_idx):
    raise NotImplementedError("write your pallas kernel here")


# speedup vs baseline: 1.2901x; 1.2901x over previous
import functools

import jax
import jax.numpy as jnp
from jax import lax
from jax.experimental import pallas as pl
from jax.experimental.pallas import tpu as pltpu
from jax.experimental.pallas import tpu_sc as plsc

_B = 1024
_D = 16
_N = 1000000
_TEMP = 0.07
_MOM = 0.5
_EPS = 1e-12

_R = 4000
_NC = 2
_NS = 16
_NW = _NC * _NS
_RPW = _B // _NW


def _tc_body(f_ref, g_ref, bank_ref, bankout_ref, gn_ref, loss_ref,
             fn_ref, acc_ref):
    i = pl.program_id(0)
    nb = pl.num_programs(0)

    @pl.when(i == 0)
    def _init():
        f = f_ref[...]
        ss = jnp.sum(f * f, axis=1, keepdims=True)
        fn_ref[...] = f / jnp.maximum(jnp.sqrt(ss), _EPS)
        acc_ref[...] = jnp.zeros_like(acc_ref)

    blk = bank_ref[...]
    bankout_ref[...] = blk
    s = lax.dot_general(fn_ref[...], blk, (((1,), (1,)), ((), ())),
                        preferred_element_type=jnp.float32)
    e = jnp.exp((s - 1.0) * (1.0 / _TEMP))
    acc_ref[...] += jnp.sum(e, axis=1)

    @pl.when(i == nb - 1)
    def _fin():
        g = g_ref[...]
        gss = jnp.sum(g * g, axis=1, keepdims=True)
        gn = g / jnp.maximum(jnp.sqrt(gss), _EPS)
        gn_ref[...] = gn
        pos = jnp.sum(fn_ref[...] * gn, axis=1)
        total = acc_ref[...] + jnp.exp((pos - 1.0) * (1.0 / _TEMP))
        lvec = (1.0 / _TEMP) + jnp.log(total) - pos * (1.0 / _TEMP)
        loss_ref[...] = jnp.broadcast_to(jnp.mean(lvec), (1, 1))


def _rsqrt_newton(x):
    xi = lax.bitcast_convert_type(x, jnp.int32)
    yi = jnp.int32(0x5F3759DF) - lax.shift_right_logical(xi, 1)
    y = lax.bitcast_convert_type(yi, jnp.float32)
    for _ in range(3):
        y = y * (1.5 - 0.5 * x * y * y)
    return y


def _sc_body(gn_hbm, idx_hbm, bank_ref, idx_v, gn_v, old_v, new_v, sem):
    wid = lax.axis_index("s") * _NC + lax.axis_index("c")
    base = wid * _RPW
    pltpu.sync_copy(idx_hbm.at[pl.ds(base, _RPW)], idx_v)
    pltpu.sync_copy(gn_hbm.at[pl.ds(base, _RPW)], gn_v)
    pltpu.async_copy(bank_ref.at[idx_v], old_v, sem).wait()
    for grp in range(_RPW // 16):
        rows = lax.iota(jnp.int32, 16) + jnp.int32(grp * 16)
        vs = []
        acc = jnp.zeros((16,), jnp.float32)
        for d in range(_D):
            dcol = jnp.full((16,), d, jnp.int32)
            v = (plsc.load_gather(old_v, [rows, dcol]) * _MOM
                 + plsc.load_gather(gn_v, [rows, dcol]) * (1.0 - _MOM))
            vs.append(v)
            acc += v * v
        rs = _rsqrt_newton(acc)
        for d in range(_D):
            dcol = jnp.full((16,), d, jnp.int32)
            plsc.store_scatter(new_v, [rows, dcol], vs[d] * rs)
    pltpu.async_copy(new_v, bank_ref.at[idx_v], sem).wait()


@functools.cache
def _sc_update():
    return pl.kernel(
        _sc_body,
        out_type=(),
        mesh=plsc.VectorSubcoreMesh(core_axis_name="c", subcore_axis_name="s",
                                    num_cores=_NC, num_subcores=_NS),
        compiler_params=pltpu.CompilerParams(needs_layout_passes=False,
                                             use_tc_tiling_on_sc=False),
        scratch_types=[
            pltpu.VMEM((_RPW,), jnp.int32),
            pltpu.VMEM((_RPW, _D), jnp.float32),
            pltpu.VMEM((_RPW, _D), jnp.float32),
            pltpu.VMEM((_RPW, _D), jnp.float32),
            pltpu.SemaphoreType.DMA,
        ],
    )


_tc_pass = pl.pallas_call(
    _tc_body,
    grid=(_N // _R,),
    in_specs=[
        pl.BlockSpec((_B, _D), lambda i: (0, 0)),
        pl.BlockSpec((_B, _D), lambda i: (0, 0)),
        pl.BlockSpec((_R, _D), lambda i: (i, 0)),
    ],
    out_specs=[
        pl.BlockSpec((_R, _D), lambda i: (i, 0)),
        pl.BlockSpec((_B, _D), lambda i: (0, 0)),
        pl.BlockSpec((1, 1), lambda i: (0, 0)),
    ],
    out_shape=[
        jax.ShapeDtypeStruct((_N, _D), jnp.float32),
        jax.ShapeDtypeStruct((_B, _D), jnp.float32),
        jax.ShapeDtypeStruct((1, 1), jnp.float32),
    ],
    scratch_shapes=[
        pltpu.VMEM((_B, _D), jnp.float32),
        pltpu.VMEM((_B,), jnp.float32),
    ],
)


def kernel(f, g, memory_bank, update_idx):
    bank_copy, gn, loss11 = _tc_pass(f, g, memory_bank)
    bank_r = jax.new_ref(bank_copy)
    _sc_update()(gn, update_idx, bank_r)
    return loss11[0, 0], bank_r[...]


# speedup vs baseline: 1.5777x; 1.2229x over previous
import functools

import jax
import jax.numpy as jnp
from jax import lax
from jax.experimental import pallas as pl
from jax.experimental.pallas import tpu as pltpu
from jax.experimental.pallas import tpu_sc as plsc
from jax._src.pallas import mpmd as _mpmd

_B = 1024
_D = 16
_N = 1000000
_TEMP = 0.07
_MOM = 0.5
_EPS = 1e-12

_R = 4000
_NC = 2
_NS = 16
_NW = _NC * _NS
_RPW = _B // _NW


def _tc_body(f_ref, g_ref, bank_ref, bankout_ref, gn_ref, loss_ref,
             fnt_ref, acc_ref):
    i = pl.program_id(0)
    nb = pl.num_programs(0)

    @pl.when(i == 0)
    def _init():
        ft = f_ref[...].T
        ss = jnp.sum(ft * ft, axis=0, keepdims=True)
        fnt_ref[...] = ft / jnp.maximum(jnp.sqrt(ss), _EPS)
        acc_ref[...] = jnp.zeros_like(acc_ref)

    blk = bank_ref[...]
    bankout_ref[...] = blk
    s = lax.dot_general(blk, fnt_ref[...], (((1,), (0,)), ((), ())),
                        preferred_element_type=jnp.float32)
    e = jnp.exp((s - 1.0) * (1.0 / _TEMP))
    acc_ref[...] += jnp.sum(e.reshape(_R // 8, 8, _B), axis=0)

    @pl.when(i == nb - 1)
    def _fin():
        gt = g_ref[...].T
        gss = jnp.sum(gt * gt, axis=0, keepdims=True)
        gnt = gt / jnp.maximum(jnp.sqrt(gss), _EPS)
        gn_ref[...] = gnt.T
        pos = jnp.sum(fnt_ref[...] * gnt, axis=0)
        total = jnp.sum(acc_ref[...], axis=0) \
            + jnp.exp((pos - 1.0) * (1.0 / _TEMP))
        lvec = (1.0 / _TEMP) + jnp.log(total) - pos * (1.0 / _TEMP)
        loss_ref[...] = jnp.broadcast_to(jnp.mean(lvec), (1, 1))


def _rsqrt_newton(x):
    xi = lax.bitcast_convert_type(x, jnp.int32)
    yi = jnp.int32(0x5F3759DF) - lax.shift_right_logical(xi, 1)
    y = lax.bitcast_convert_type(yi, jnp.float32)
    for _ in range(3):
        y = y * (1.5 - 0.5 * x * y * y)
    return y


def _sc_body(gn_hbm, idx_hbm, bank_in, bank_out, idx_v, gn_v, old_v, new_v,
             sem):
    wid = lax.axis_index("s") * _NC + lax.axis_index("c")
    base = wid * _RPW
    pltpu.sync_copy(idx_hbm.at[pl.ds(base, _RPW)], idx_v)
    pltpu.sync_copy(gn_hbm.at[pl.ds(base, _RPW)], gn_v)
    pltpu.async_copy(bank_in.at[idx_v], old_v, sem).wait()
    for grp in range(_RPW // 16):
        rows = lax.iota(jnp.int32, 16) + jnp.int32(grp * 16)
        vs = []
        acc = jnp.zeros((16,), jnp.float32)
        for d in range(_D):
            dcol = jnp.full((16,), d, jnp.int32)
            v = (plsc.load_gather(old_v, [rows, dcol]) * _MOM
                 + plsc.load_gather(gn_v, [rows, dcol]) * (1.0 - _MOM))
            vs.append(v)
            acc += v * v
        rs = _rsqrt_newton(acc)
        for d in range(_D):
            dcol = jnp.full((16,), d, jnp.int32)
            plsc.store_scatter(new_v, [rows, dcol], vs[d] * rs)
    pltpu.async_copy(new_v, bank_out.at[idx_v], sem).wait()


@functools.cache
def _sc_update():
    mesh = plsc.VectorSubcoreMesh(core_axis_name="c", subcore_axis_name="s",
                                  num_cores=_NC, num_subcores=_NS)
    return _mpmd._mpmd_map(
        [(mesh, _sc_body)],
        [jax.ShapeDtypeStruct((_N, _D), jnp.float32)],
        input_output_aliases={2: 0},
        scratch_types=[
            pltpu.VMEM((_RPW,), jnp.int32),
            pltpu.VMEM((_RPW, _D), jnp.float32),
            pltpu.VMEM((_RPW, _D), jnp.float32),
            pltpu.VMEM((_RPW, _D), jnp.float32),
            pltpu.SemaphoreType.DMA,
        ],
        compiler_params=pltpu.CompilerParams(needs_layout_passes=False,
                                             use_tc_tiling_on_sc=False),
    )


_tc_pass = pl.pallas_call(
    _tc_body,
    grid=(_N // _R,),
    in_specs=[
        pl.BlockSpec((_B, _D), lambda i: (0, 0)),
        pl.BlockSpec((_B, _D), lambda i: (0, 0)),
        pl.BlockSpec((_R, _D), lambda i: (i, 0)),
    ],
    out_specs=[
        pl.BlockSpec((_R, _D), lambda i: (i, 0)),
        pl.BlockSpec((_B, _D), lambda i: (0, 0)),
        pl.BlockSpec((1, 1), lambda i: (0, 0)),
    ],
    out_shape=[
        jax.ShapeDtypeStruct((_N, _D), jnp.float32),
        jax.ShapeDtypeStruct((_B, _D), jnp.float32),
        jax.ShapeDtypeStruct((1, 1), jnp.float32),
    ],
    scratch_shapes=[
        pltpu.VMEM((_D, _B), jnp.float32),
        pltpu.VMEM((8, _B), jnp.float32),
    ],
)


def kernel(f, g, memory_bank, update_idx):
    bank_copy, gn, loss11 = _tc_pass(f, g, memory_bank)
    (new_bank,) = _sc_update()(gn, update_idx, bank_copy)
    return loss11[0, 0], new_bank
